# baseline (speedup 1.0000x reference)
import functools

import jax
import jax.numpy as jnp
from jax import lax
from jax.experimental import pallas as pl
from jax.experimental.pallas import tpu as pltpu
from jax.experimental.pallas import tpu_sc as plsc

N = 10000
E = 320000
D = 128
NUM_ATOM_FEATS = 9
ATOM_VOCAB = 100
NUM_LAYERS = 4

NC = 2
NS = 16
NWORK = NC * NS
CH = 128
EPW = 10240
NCHUNK = EPW // CH
NCHUNK_A = NCHUNK + 3
E_PAD = EPW * NWORK
N_PAD = 10112
ZROWS = N_PAD // NS
NQ = 4
NG = 2

BN = 1000
GRID = N // BN


def _segsum_body(y_hbm, sidx_hbm, didx_hbm, zeros_hbm, out_hbm,
                 sq0, sq1, sq2, sq3, dq0, dq1, dq2, dq3, g0, g1, acc,
                 is0, is1, is2, is3, gs0, gs1, ss0, ss1):
    c = lax.axis_index("c")
    s = lax.axis_index("s")
    w = c * NS + s
    sq = (sq0, sq1, sq2, sq3)
    dq = (dq0, dq1, dq2, dq3)
    isem = (is0, is1, is2, is3)
    gbuf = (g0, g1)
    gsem = (gs0, gs1)
    ssem = (ss0, ss1)

    pltpu.sync_copy(zeros_hbm, acc.at[pl.ds(s * ZROWS, ZROWS)])
    plsc.subcore_barrier()

    def fire_idx(j, q):
        pltpu.async_copy(sidx_hbm.at[w, j], sq[q], isem[q])
        pltpu.async_copy(didx_hbm.at[w, j], dq[q], isem[q])

    def wait_idx(j, q):
        pltpu.make_async_copy(sidx_hbm.at[w, j], sq[q], isem[q]).wait()
        pltpu.make_async_copy(didx_hbm.at[w, j], dq[q], isem[q]).wait()

    def fire_gather(b, q):
        pltpu.async_copy(y_hbm.at[sq[q]], gbuf[b], gsem[b])

    def wait_gather(b, q):
        pltpu.make_async_copy(y_hbm.at[sq[q]], gbuf[b], gsem[b]).wait()

    def fire_scatter(b, q):
        pltpu.async_copy(gbuf[b], acc.at[dq[q]], ssem[b], add=True)

    def wait_scatter(b, q):
        pltpu.make_async_copy(gbuf[b], acc.at[dq[q]], ssem[b]).wait()

    def body(j, u):
        b, bn = u % NG, (u + 1) % NG
        q, qn, qf = u % NQ, (u + 1) % NQ, (u + 3) % NQ
        wait_scatter(bn, (u - 1) % NQ)

        fire_idx(j + 3, qf)
        wait_idx(j + 1, qn)
        fire_gather(bn, qn)
        wait_gather(b, q)
        fire_scatter(b, q)

    for j in range(3):
        fire_idx(j, j % NQ)
    wait_idx(0, 0)
    fire_gather(0, 0)
    fire_idx(3, 3)
    wait_idx(1, 1)
    fire_gather(1, 1)
    wait_gather(0, 0)
    fire_scatter(0, 0)
    for j in range(1, 4):
        body(j, j)

    def outer(t, carry):
        j0 = 4 + t * 4
        for u in range(4):
            body(j0 + u, u)
        return carry

    lax.fori_loop(0, (NCHUNK - 4) // 4, outer, 0)
    wait_scatter((NCHUNK - 1) % NG, (NCHUNK - 1) % NQ)
    wait_gather(NCHUNK % NG, NCHUNK % NQ)
    wait_idx(NCHUNK + 1, (NCHUNK + 1) % NQ)
    wait_idx(NCHUNK + 2, (NCHUNK + 2) % NQ)

    plsc.subcore_barrier()
    pltpu.sync_copy(acc.at[pl.ds(s * ZROWS, ZROWS)],
                    out_hbm.at[c, pl.ds(s * ZROWS, ZROWS)])


assert (NCHUNK - 4) % 4 == 0

_segsum = functools.partial(
    pl.kernel,
    out_type=jax.ShapeDtypeStruct((NC, N_PAD, D), jnp.float32),
    mesh=plsc.VectorSubcoreMesh(core_axis_name="c", subcore_axis_name="s",
                                num_cores=NC, num_subcores=NS),
    compiler_params=pltpu.CompilerParams(use_tc_tiling_on_sc=False),
    scratch_types=(
        [pltpu.VMEM((CH,), jnp.int32)] * 4
        + [pltpu.VMEM((CH,), jnp.int32)] * 4
        + [pltpu.VMEM((CH, D), jnp.float32)] * 2
        + [pltpu.VMEM_SHARED((N_PAD, D), jnp.float32)]
        + [pltpu.SemaphoreType.DMA] * 8
    ),
)(_segsum_body)


def _encoder_body(h_ref, tab_ref, degp_ref, x0_ref, y0_ref, dmat_ref):
    acc = jnp.zeros((BN, D), jnp.float32)
    iota = lax.broadcasted_iota(jnp.int32, (BN, ATOM_VOCAB), 1)
    for f in range(NUM_ATOM_FEATS):
        col = h_ref[:, f].reshape(BN, 1)
        oh = (col == iota).astype(jnp.float32)
        acc = acc + jnp.dot(oh, tab_ref[f], preferred_element_type=jnp.float32)
    deg = degp_ref[0] + degp_ref[1]
    dmat = lax.rsqrt(jnp.maximum(deg, 1.0))
    x0_ref[...] = acc
    dmat_ref[...] = dmat
    y0_ref[...] = acc * dmat


def _encoder(hm, tables, degp):
    return pl.pallas_call(
        _encoder_body,
        grid=(GRID,),
        in_specs=[
            pl.BlockSpec((BN, NUM_ATOM_FEATS), lambda i: (i, 0)),
            pl.BlockSpec((NUM_ATOM_FEATS, ATOM_VOCAB, D), lambda i: (0, 0, 0)),
            pl.BlockSpec((NC, BN, D), lambda i: (0, i, 0)),
        ],
        out_specs=[
            pl.BlockSpec((BN, D), lambda i: (i, 0)),
            pl.BlockSpec((BN, D), lambda i: (i, 0)),
            pl.BlockSpec((BN, D), lambda i: (i, 0)),
        ],
        out_shape=[jax.ShapeDtypeStruct((N, D), jnp.float32)] * 3,
    )(hm, tables, degp)


def _stage_a_body(s1p_ref, dmat_ref, x1_ref, y1_ref):
    dmat = dmat_ref[...]
    x1 = -((s1p_ref[0] + s1p_ref[1]) * dmat)
    x1_ref[...] = x1
    y1_ref[...] = x1 * dmat


def _stage_a(s1p, dmat):
    return pl.pallas_call(
        _stage_a_body,
        grid=(GRID,),
        in_specs=[
            pl.BlockSpec((NC, BN, D), lambda i: (0, i, 0)),
            pl.BlockSpec((BN, D), lambda i: (i, 0)),
        ],
        out_specs=[
            pl.BlockSpec((BN, D), lambda i: (i, 0)),
            pl.BlockSpec((BN, D), lambda i: (i, 0)),
        ],
        out_shape=[jax.ShapeDtypeStruct((N, D), jnp.float32)] * 2,
    )(s1p, dmat)


def _stage_b1_body(s2p_ref, dmat_ref, x0_ref, x1_ref, w_ref, hh_ref, stats_ref):
    i = pl.program_id(0)
    x0 = x0_ref[...]
    x2 = -2.0 * ((s2p_ref[0] + s2p_ref[1]) * dmat_ref[...]) - x0
    hh = (jnp.dot(x0, w_ref[0], preferred_element_type=jnp.float32)
          + jnp.dot(x1_ref[...], w_ref[1], preferred_element_type=jnp.float32)
          + jnp.dot(x2, w_ref[2], preferred_element_type=jnp.float32))
    hh_ref[...] = hh
    ssum = jnp.sum(hh, axis=0, keepdims=True)
    ssq = jnp.sum(hh * hh, axis=0, keepdims=True)
    upd = jnp.concatenate(
        [ssum, ssq, jnp.zeros((6, D), jnp.float32)], axis=0)

    @pl.when(i == 0)
    def _():
        stats_ref[...] = upd

    @pl.when(i > 0)
    def _():
        stats_ref[...] = stats_ref[...] + upd


def _stage_b1(s2p, dmat, x0, x1, wl):
    return pl.pallas_call(
        _stage_b1_body,
        grid=(GRID,),
        in_specs=[
            pl.BlockSpec((NC, BN, D), lambda i: (0, i, 0)),
            pl.BlockSpec((BN, D), lambda i: (i, 0)),
            pl.BlockSpec((BN, D), lambda i: (i, 0)),
            pl.BlockSpec((BN, D), lambda i: (i, 0)),
            pl.BlockSpec((3, D, D), lambda i: (0, 0, 0)),
        ],
        out_specs=[
            pl.BlockSpec((BN, D), lambda i: (i, 0)),
            pl.BlockSpec((8, D), lambda i: (0, 0)),
        ],
        out_shape=[
            jax.ShapeDtypeStruct((N, D), jnp.float32),
            jax.ShapeDtypeStruct((8, D), jnp.float32),
        ],
    )(s2p, dmat, x0, x1, wl)


def _stage_b2_body(hh_ref, stats_ref, x0_ref, gb_ref, dmat_ref,
                   x_ref, y_ref, xsum_ref):
    i = pl.program_id(0)
    mu = stats_ref[0:1, :] * (1.0 / N)
    var = stats_ref[1:2, :] * (1.0 / N) - mu * mu
    rstd = lax.rsqrt(var + 1e-5)
    hn = (hh_ref[...] - mu) * rstd * gb_ref[0:1, :] + gb_ref[1:2, :]
    x = x0_ref[...] + jnp.maximum(hn, 0.0)
    x_ref[...] = x
    y_ref[...] = x * dmat_ref[...]
    upd = jnp.concatenate(
        [jnp.sum(x, axis=0, keepdims=True), jnp.zeros((7, D), jnp.float32)],
        axis=0)

    @pl.when(i == 0)
    def _():
        xsum_ref[...] = upd

    @pl.when(i > 0)
    def _():
        xsum_ref[...] = xsum_ref[...] + upd


def _stage_b2(hh, stats, x0, gb, dmat):
    return pl.pallas_call(
        _stage_b2_body,
        grid=(GRID,),
        in_specs=[
            pl.BlockSpec((BN, D), lambda i: (i, 0)),
            pl.BlockSpec((8, D), lambda i: (0, 0)),
            pl.BlockSpec((BN, D), lambda i: (i, 0)),
            pl.BlockSpec((8, D), lambda i: (0, 0)),
            pl.BlockSpec((BN, D), lambda i: (i, 0)),
        ],
        out_specs=[
            pl.BlockSpec((BN, D), lambda i: (i, 0)),
            pl.BlockSpec((BN, D), lambda i: (i, 0)),
            pl.BlockSpec((8, D), lambda i: (0, 0)),
        ],
        out_shape=[
            jax.ShapeDtypeStruct((N, D), jnp.float32),
            jax.ShapeDtypeStruct((N, D), jnp.float32),
            jax.ShapeDtypeStruct((8, D), jnp.float32),
        ],
    )(hh, stats, x0, gb, dmat)


def _readout_body(xsum_ref, w1_ref, b1_ref, w2_ref, b2_ref, w3_ref, b3_ref,
                  y_ref):
    hg = xsum_ref[0:1, :] * (1.0 / N)
    y1 = jnp.maximum(
        jnp.dot(hg, w1_ref[...], preferred_element_type=jnp.float32)
        + b1_ref[...], 0.0)
    y2 = jnp.maximum(
        jnp.dot(y1, w2_ref[...], preferred_element_type=jnp.float32)
        + b2_ref[...], 0.0)
    y_ref[...] = (jnp.dot(y2, w3_ref[...], preferred_element_type=jnp.float32)
                  + b3_ref[...])


def _readout(xsum, w1, b1, w2, b2, w3, b3):
    return pl.pallas_call(
        _readout_body,
        out_shape=jax.ShapeDtypeStruct((1, b3.shape[-1]), jnp.float32),
    )(xsum, w1, b1, w2, b2, w3, b3)


def kernel(h, e, edge_index, atom_tables, bond_tables, W, gamma, beta,
           rW1, rb1, rW2, rb2, rW3, rb3):
    del e, bond_tables
    src = edge_index[0].astype(jnp.int32)
    dst = edge_index[1].astype(jnp.int32)
    pad = E_PAD - E
    sidxp = jnp.concatenate([src, jnp.zeros((pad,), jnp.int32)])
    sidxp = jnp.concatenate(
        [sidxp.reshape(NWORK, NCHUNK, CH), jnp.zeros((NWORK, 3, CH), jnp.int32)],
        axis=1)
    didxp = jnp.concatenate([dst, jnp.full((pad,), N, jnp.int32)])
    didxp = jnp.concatenate(
        [didxp.reshape(NWORK, NCHUNK, CH),
         jnp.full((NWORK, 3, CH), N, jnp.int32)],
        axis=1)
    zrows = jnp.zeros((ZROWS, D), jnp.float32)

    degp = _segsum(jnp.ones((N, D), jnp.float32), sidxp, didxp, zrows)
    x0, y, dmat = _encoder(h.astype(jnp.int32), atom_tables, degp)

    xsum = None
    for l in range(NUM_LAYERS):
        s1p = _segsum(y, sidxp, didxp, zrows)
        x1, y1 = _stage_a(s1p, dmat)
        s2p = _segsum(y1, sidxp, didxp, zrows)
        hh, stats = _stage_b1(s2p, dmat, x0, x1, W[l].reshape(3, D, D))
        gb = jnp.concatenate(
            [gamma[l].reshape(1, D), beta[l].reshape(1, D),
             jnp.zeros((6, D), jnp.float32)], axis=0)
        x0, y, xsum = _stage_b2(hh, stats, x0, gb, dmat)

    return _readout(xsum, rW1, rb1.reshape(1, -1), rW2, rb2.reshape(1, -1),
                    rW3, rb3.reshape(1, -1))


# speedup vs baseline: 1.9436x; 1.9436x over previous
import functools

import jax
import jax.numpy as jnp
from jax import lax
from jax.experimental import pallas as pl
from jax.experimental.pallas import tpu as pltpu
from jax.experimental.pallas import tpu_sc as plsc

N = 10000
E = 320000
D = 128
DH = D // 2
NUM_ATOM_FEATS = 9
ATOM_VOCAB = 100
NUM_LAYERS = 4

NC = 2
NS = 16
CH = 256
EPT = E // NS
NCHUNK = 80
NCHUNK_A = NCHUNK + 1
EPT_A = NCHUNK_A * CH
N_PAD = 10112
ZROWS = N_PAD // NS

BN = 1000
GRID = N // BN


NSC = NCHUNK


def _segsum_body(y_hbm, sidx_hbm, didx_hbm, zeros_hbm, out_hbm,
                 sidx_v, didx_v, g0, g1, acc, gs0, gs1, ss0, ss1):
    c = lax.axis_index("c")
    s = lax.axis_index("s")
    gbufs = (g0, g1)
    gsems = (gs0, gs1)
    ssems = (ss0, ss1)

    pltpu.sync_copy(zeros_hbm, acc.at[pl.ds(s * ZROWS, ZROWS)])
    pltpu.sync_copy(sidx_hbm.at[s], sidx_v)
    pltpu.sync_copy(didx_hbm.at[s], didx_v)
    plsc.subcore_barrier()

    yc = y_hbm.at[c]

    def fire_gather(j, b):
        pltpu.async_copy(yc.at[sidx_v.at[j]], gbufs[b], gsems[b])

    def wait_gather(j, b):
        pltpu.make_async_copy(yc.at[sidx_v.at[j]], gbufs[b], gsems[b]).wait()

    def fire_scatter(j, b):
        pltpu.async_copy(gbufs[b], acc.at[didx_v.at[j]], ssems[b], add=True)

    def wait_scatter(j, b):
        pltpu.make_async_copy(gbufs[b], acc.at[didx_v.at[j]],
                              ssems[b]).wait()

    def body(j, b):
        bn = (b + 1) % 2
        wait_scatter(j - 1, bn)
        fire_gather(j + 1, bn)
        wait_gather(j, b)
        fire_scatter(j, b)

    fire_gather(0, 0)
    fire_gather(1, 1)
    wait_gather(0, 0)
    fire_scatter(0, 0)
    body(1, 1)

    def outer(j0, carry):
        for u in range(2):
            body(j0 + u, u)
        return carry

    lax.fori_loop(0, (NSC - 2) // 2, lambda t, cr: outer(2 + t * 2, cr), 0)
    wait_scatter(NSC - 1, (NSC - 1) % 2)
    wait_gather(NSC, NSC % 2)

    plsc.subcore_barrier()
    pltpu.sync_copy(acc.at[pl.ds(s * ZROWS, ZROWS)],
                    out_hbm.at[c, pl.ds(s * ZROWS, ZROWS)])


assert (NSC - 2) % 2 == 0 and NCHUNK_A >= NSC + 1

_segsum = functools.partial(
    pl.kernel,
    out_type=jax.ShapeDtypeStruct((NC, N_PAD, DH), jnp.float32),
    mesh=plsc.VectorSubcoreMesh(core_axis_name="c", subcore_axis_name="s",
                                num_cores=NC, num_subcores=NS),
    compiler_params=pltpu.CompilerParams(use_tc_tiling_on_sc=False),
    scratch_types=[
        pltpu.VMEM((NCHUNK_A, CH), jnp.int32),
        pltpu.VMEM((NCHUNK_A, CH), jnp.int32),
        pltpu.VMEM((CH, DH), jnp.float32),
        pltpu.VMEM((CH, DH), jnp.float32),
        pltpu.VMEM_SHARED((N_PAD, DH), jnp.float32),
        pltpu.SemaphoreType.DMA,
        pltpu.SemaphoreType.DMA,
        pltpu.SemaphoreType.DMA,
        pltpu.SemaphoreType.DMA,
    ],
)(_segsum_body)


def _split(y):
    return jnp.stack([y[:, :DH], y[:, DH:]])


def _encoder_body(h_ref, tab_ref, degp_ref, x0_ref, y0_ref, dmat_ref):
    acc = jnp.zeros((BN, D), jnp.float32)
    iota = lax.broadcasted_iota(jnp.int32, (BN, ATOM_VOCAB), 1)
    for f in range(NUM_ATOM_FEATS):
        col = h_ref[:, f].reshape(BN, 1)
        oh = (col == iota).astype(jnp.float32)
        acc = acc + jnp.dot(oh, tab_ref[f], preferred_element_type=jnp.float32)
    deg = jnp.concatenate([degp_ref[0], degp_ref[1]], axis=1)
    dmat = lax.rsqrt(jnp.maximum(deg, 1.0))
    x0_ref[...] = acc
    dmat_ref[...] = dmat
    y0_ref[...] = _split(acc * dmat)


def _encoder(hm, tables, degp):
    return pl.pallas_call(
        _encoder_body,
        grid=(GRID,),
        in_specs=[
            pl.BlockSpec((BN, NUM_ATOM_FEATS), lambda i: (i, 0)),
            pl.BlockSpec((NUM_ATOM_FEATS, ATOM_VOCAB, D), lambda i: (0, 0, 0)),
            pl.BlockSpec((NC, BN, DH), lambda i: (0, i, 0)),
        ],
        out_specs=[
            pl.BlockSpec((BN, D), lambda i: (i, 0)),
            pl.BlockSpec((NC, BN, DH), lambda i: (0, i, 0)),
            pl.BlockSpec((BN, D), lambda i: (i, 0)),
        ],
        out_shape=[
            jax.ShapeDtypeStruct((N, D), jnp.float32),
            jax.ShapeDtypeStruct((NC, N, DH), jnp.float32),
            jax.ShapeDtypeStruct((N, D), jnp.float32),
        ],
    )(hm, tables, degp)


def _stage_a_body(s1p_ref, dmat_ref, x1_ref, y1_ref):
    dmat = dmat_ref[...]
    s1 = jnp.concatenate([s1p_ref[0], s1p_ref[1]], axis=1)
    x1 = -(s1 * dmat)
    x1_ref[...] = x1
    y1_ref[...] = _split(x1 * dmat)


def _stage_a(s1p, dmat):
    return pl.pallas_call(
        _stage_a_body,
        grid=(GRID,),
        in_specs=[
            pl.BlockSpec((NC, BN, DH), lambda i: (0, i, 0)),
            pl.BlockSpec((BN, D), lambda i: (i, 0)),
        ],
        out_specs=[
            pl.BlockSpec((BN, D), lambda i: (i, 0)),
            pl.BlockSpec((NC, BN, DH), lambda i: (0, i, 0)),
        ],
        out_shape=[
            jax.ShapeDtypeStruct((N, D), jnp.float32),
            jax.ShapeDtypeStruct((NC, N, DH), jnp.float32),
        ],
    )(s1p, dmat)


def _stage_b1_body(s2p_ref, dmat_ref, x0_ref, x1_ref, w_ref, hh_ref, stats_ref):
    i = pl.program_id(0)
    x0 = x0_ref[...]
    s2 = jnp.concatenate([s2p_ref[0], s2p_ref[1]], axis=1)
    x2 = -2.0 * (s2 * dmat_ref[...]) - x0
    hh = (jnp.dot(x0, w_ref[0], preferred_element_type=jnp.float32)
          + jnp.dot(x1_ref[...], w_ref[1], preferred_element_type=jnp.float32)
          + jnp.dot(x2, w_ref[2], preferred_element_type=jnp.float32))
    hh_ref[...] = hh
    ssum = jnp.sum(hh, axis=0, keepdims=True)
    ssq = jnp.sum(hh * hh, axis=0, keepdims=True)
    upd = jnp.concatenate(
        [ssum, ssq, jnp.zeros((6, D), jnp.float32)], axis=0)

    @pl.when(i == 0)
    def _():
        stats_ref[...] = upd

    @pl.when(i > 0)
    def _():
        stats_ref[...] = stats_ref[...] + upd


def _stage_b1(s2p, dmat, x0, x1, wl):
    return pl.pallas_call(
        _stage_b1_body,
        grid=(GRID,),
        in_specs=[
            pl.BlockSpec((NC, BN, DH), lambda i: (0, i, 0)),
            pl.BlockSpec((BN, D), lambda i: (i, 0)),
            pl.BlockSpec((BN, D), lambda i: (i, 0)),
            pl.BlockSpec((BN, D), lambda i: (i, 0)),
            pl.BlockSpec((3, D, D), lambda i: (0, 0, 0)),
        ],
        out_specs=[
            pl.BlockSpec((BN, D), lambda i: (i, 0)),
            pl.BlockSpec((8, D), lambda i: (0, 0)),
        ],
        out_shape=[
            jax.ShapeDtypeStruct((N, D), jnp.float32),
            jax.ShapeDtypeStruct((8, D), jnp.float32),
        ],
    )(s2p, dmat, x0, x1, wl)


def _stage_b2_body(hh_ref, stats_ref, x0_ref, gb_ref, dmat_ref,
                   x_ref, y_ref, xsum_ref):
    i = pl.program_id(0)
    mu = stats_ref[0:1, :] * (1.0 / N)
    var = stats_ref[1:2, :] * (1.0 / N) - mu * mu
    rstd = lax.rsqrt(var + 1e-5)
    hn = (hh_ref[...] - mu) * rstd * gb_ref[0:1, :] + gb_ref[1:2, :]
    x = x0_ref[...] + jnp.maximum(hn, 0.0)
    x_ref[...] = x
    y_ref[...] = _split(x * dmat_ref[...])
    upd = jnp.concatenate(
        [jnp.sum(x, axis=0, keepdims=True), jnp.zeros((7, D), jnp.float32)],
        axis=0)

    @pl.when(i == 0)
    def _():
        xsum_ref[...] = upd

    @pl.when(i > 0)
    def _():
        xsum_ref[...] = xsum_ref[...] + upd


def _stage_b2(hh, stats, x0, gb, dmat):
    return pl.pallas_call(
        _stage_b2_body,
        grid=(GRID,),
        in_specs=[
            pl.BlockSpec((BN, D), lambda i: (i, 0)),
            pl.BlockSpec((8, D), lambda i: (0, 0)),
            pl.BlockSpec((BN, D), lambda i: (i, 0)),
            pl.BlockSpec((8, D), lambda i: (0, 0)),
            pl.BlockSpec((BN, D), lambda i: (i, 0)),
        ],
        out_specs=[
            pl.BlockSpec((BN, D), lambda i: (i, 0)),
            pl.BlockSpec((NC, BN, DH), lambda i: (0, i, 0)),
            pl.BlockSpec((8, D), lambda i: (0, 0)),
        ],
        out_shape=[
            jax.ShapeDtypeStruct((N, D), jnp.float32),
            jax.ShapeDtypeStruct((NC, N, DH), jnp.float32),
            jax.ShapeDtypeStruct((8, D), jnp.float32),
        ],
    )(hh, stats, x0, gb, dmat)


def _readout_body(xsum_ref, w1_ref, b1_ref, w2_ref, b2_ref, w3_ref, b3_ref,
                  y_ref):
    hg = xsum_ref[0:1, :] * (1.0 / N)
    y1 = jnp.maximum(
        jnp.dot(hg, w1_ref[...], preferred_element_type=jnp.float32)
        + b1_ref[...], 0.0)
    y2 = jnp.maximum(
        jnp.dot(y1, w2_ref[...], preferred_element_type=jnp.float32)
        + b2_ref[...], 0.0)
    y_ref[...] = (jnp.dot(y2, w3_ref[...], preferred_element_type=jnp.float32)
                  + b3_ref[...])


def _readout(xsum, w1, b1, w2, b2, w3, b3):
    return pl.pallas_call(
        _readout_body,
        out_shape=jax.ShapeDtypeStruct((1, b3.shape[-1]), jnp.float32),
    )(xsum, w1, b1, w2, b2, w3, b3)


def kernel(h, e, edge_index, atom_tables, bond_tables, W, gamma, beta,
           rW1, rb1, rW2, rb2, rW3, rb3):
    del e, bond_tables
    src = edge_index[0].astype(jnp.int32)
    dst = edge_index[1].astype(jnp.int32)
    sidxp = jnp.pad(src.reshape(NS, EPT),
                    ((0, 0), (0, EPT_A - EPT))).reshape(NS, NCHUNK_A, CH)
    didxp = jnp.pad(dst.reshape(NS, EPT), ((0, 0), (0, EPT_A - EPT)),
                    constant_values=N).reshape(NS, NCHUNK_A, CH)
    zrows = jnp.zeros((ZROWS, DH), jnp.float32)

    degp = _segsum(jnp.ones((NC, N, DH), jnp.float32), sidxp, didxp, zrows)
    x0, y, dmat = _encoder(h.astype(jnp.int32), atom_tables, degp)

    xsum = None
    for l in range(NUM_LAYERS):
        s1p = _segsum(y, sidxp, didxp, zrows)
        x1, y1 = _stage_a(s1p, dmat)
        s2p = _segsum(y1, sidxp, didxp, zrows)
        hh, stats = _stage_b1(s2p, dmat, x0, x1, W[l].reshape(3, D, D))
        gb = jnp.concatenate(
            [gamma[l].reshape(1, D), beta[l].reshape(1, D),
             jnp.zeros((6, D), jnp.float32)], axis=0)
        x0, y, xsum = _stage_b2(hh, stats, x0, gb, dmat)

    return _readout(xsum, rW1, rb1.reshape(1, -1), rW2, rb2.reshape(1, -1),
                    rW3, rb3.reshape(1, -1))


# speedup vs baseline: 2.1114x; 1.0863x over previous
import functools

import jax
import jax.numpy as jnp
from jax import lax
from jax.experimental import pallas as pl
from jax.experimental.pallas import tpu as pltpu
from jax.experimental.pallas import tpu_sc as plsc

N = 10000
E = 320000
D = 128
DH = D // 2
NUM_ATOM_FEATS = 9
ATOM_VOCAB = 100
NUM_LAYERS = 4

NC = 2
NS = 16
CH = 128
EPT = E // NS
NCHUNK = 160
NCHUNK_A = NCHUNK + 2
EPT_A = NCHUNK_A * CH
N_PAD = 10112
ZROWS = N_PAD // NS

BN = 1000
GRID = N // BN


RING = 4
FIRE = 2


def _segsum_body(y_hbm, sidx_hbm, didx_hbm, zeros_hbm, out_hbm,
                 sidx_v, didx_v, g0, g1, g2, g3, acc,
                 gs0, gs1, gs2, gs3, ss0, ss1, ss2, ss3):
    c = lax.axis_index("c")
    s = lax.axis_index("s")
    gbufs = (g0, g1, g2, g3)
    gsems = (gs0, gs1, gs2, gs3)
    ssems = (ss0, ss1, ss2, ss3)

    pltpu.sync_copy(zeros_hbm, acc.at[pl.ds(s * ZROWS, ZROWS)])
    pltpu.sync_copy(sidx_hbm.at[s], sidx_v)
    pltpu.sync_copy(didx_hbm.at[s], didx_v)
    plsc.subcore_barrier()

    yc = y_hbm.at[c]

    def fire_gather(j, b):
        pltpu.async_copy(yc.at[sidx_v.at[j]], gbufs[b], gsems[b])

    def wait_gather(j, b):
        pltpu.make_async_copy(yc.at[sidx_v.at[j]], gbufs[b], gsems[b]).wait()

    def fire_scatter(j, b):
        pltpu.async_copy(gbufs[b], acc.at[didx_v.at[j]], ssems[b], add=True)

    def wait_scatter(j, b):
        pltpu.make_async_copy(gbufs[b], acc.at[didx_v.at[j]],
                              ssems[b]).wait()

    def body(j, b):
        bf = (b + FIRE) % RING
        wait_scatter(j - FIRE, bf)
        fire_gather(j + FIRE, bf)
        wait_gather(j, b)
        fire_scatter(j, b)

    for j in range(FIRE):
        fire_gather(j, j % RING)
    for j in range(FIRE):
        fire_gather(j + FIRE, (j + FIRE) % RING)
        wait_gather(j, j % RING)
        fire_scatter(j, j % RING)
    for j in range(FIRE, 2 * FIRE):
        body(j, j % RING)

    def outer(j0, carry):
        for u in range(RING):
            body(j0 + u, u)
        return carry

    lax.fori_loop(0, (NCHUNK - 2 * FIRE) // RING,
                  lambda t, cr: outer(2 * FIRE + t * RING, cr), 0)
    for j in range(NCHUNK - FIRE, NCHUNK):
        wait_scatter(j, j % RING)
    for j in range(NCHUNK, NCHUNK + FIRE):
        wait_gather(j, j % RING)

    plsc.subcore_barrier()
    pltpu.sync_copy(acc.at[pl.ds(s * ZROWS, ZROWS)],
                    out_hbm.at[c, pl.ds(s * ZROWS, ZROWS)])


assert (NCHUNK - 2 * FIRE) % RING == 0

_segsum = functools.partial(
    pl.kernel,
    out_type=jax.ShapeDtypeStruct((NC, N_PAD, DH), jnp.float32),
    mesh=plsc.VectorSubcoreMesh(core_axis_name="c", subcore_axis_name="s",
                                num_cores=NC, num_subcores=NS),
    compiler_params=pltpu.CompilerParams(use_tc_tiling_on_sc=False),
    scratch_types=[
        pltpu.VMEM((NCHUNK_A, CH), jnp.int32),
        pltpu.VMEM((NCHUNK_A, CH), jnp.int32),
        pltpu.VMEM((CH, DH), jnp.float32),
        pltpu.VMEM((CH, DH), jnp.float32),
        pltpu.VMEM((CH, DH), jnp.float32),
        pltpu.VMEM((CH, DH), jnp.float32),
        pltpu.VMEM_SHARED((N_PAD, DH), jnp.float32),
        pltpu.SemaphoreType.DMA,
        pltpu.SemaphoreType.DMA,
        pltpu.SemaphoreType.DMA,
        pltpu.SemaphoreType.DMA,
        pltpu.SemaphoreType.DMA,
        pltpu.SemaphoreType.DMA,
        pltpu.SemaphoreType.DMA,
        pltpu.SemaphoreType.DMA,
    ],
)(_segsum_body)


def _degsum_body(ones_hbm, didx_hbm, zeros_hbm, out_hbm,
                 didx_v, g0, acc, ss0, ss1):
    c = lax.axis_index("c")
    s = lax.axis_index("s")
    ssems = (ss0, ss1)
    pltpu.sync_copy(zeros_hbm, acc.at[pl.ds(s * ZROWS, ZROWS)])
    pltpu.sync_copy(didx_hbm.at[s], didx_v)
    pltpu.sync_copy(ones_hbm, g0)
    plsc.subcore_barrier()

    def fire(j, b):
        pltpu.async_copy(g0, acc.at[didx_v.at[j]], ssems[b], add=True)

    def wt(j, b):
        pltpu.make_async_copy(g0, acc.at[didx_v.at[j]], ssems[b]).wait()

    fire(0, 0)
    fire(1, 1)

    def outer(j0, carry):
        for u in range(2):
            j = j0 + u
            wt(j - 2, u)
            fire(j, u)
        return carry

    lax.fori_loop(0, (NCHUNK - 2) // 2, lambda t, cr: outer(2 + 2 * t, cr), 0)
    wt(NCHUNK - 2, 0)
    wt(NCHUNK - 1, 1)
    plsc.subcore_barrier()
    pltpu.sync_copy(acc.at[pl.ds(s * ZROWS, ZROWS)],
                    out_hbm.at[c, pl.ds(s * ZROWS, ZROWS)])


_degsum = functools.partial(
    pl.kernel,
    out_type=jax.ShapeDtypeStruct((NC, N_PAD, DH), jnp.float32),
    mesh=plsc.VectorSubcoreMesh(core_axis_name="c", subcore_axis_name="s",
                                num_cores=NC, num_subcores=NS),
    compiler_params=pltpu.CompilerParams(use_tc_tiling_on_sc=False),
    scratch_types=[
        pltpu.VMEM((NCHUNK_A, CH), jnp.int32),
        pltpu.VMEM((CH, DH), jnp.float32),
        pltpu.VMEM_SHARED((N_PAD, DH), jnp.float32),
        pltpu.SemaphoreType.DMA,
        pltpu.SemaphoreType.DMA,
    ],
)(_degsum_body)


def _split(y):
    return jnp.stack([y[:, :DH], y[:, DH:]])


def _encoder_body(h_ref, tab_ref, degp_ref, x0_ref, y0_ref, dmat_ref):
    acc = jnp.zeros((BN, D), jnp.float32)
    iota = lax.broadcasted_iota(jnp.int32, (BN, ATOM_VOCAB), 1)
    for f in range(NUM_ATOM_FEATS):
        col = h_ref[:, f].reshape(BN, 1)
        oh = (col == iota).astype(jnp.float32)
        acc = acc + jnp.dot(oh, tab_ref[f], preferred_element_type=jnp.float32)
    deg = jnp.concatenate([degp_ref[0], degp_ref[1]], axis=1)
    dmat = lax.rsqrt(jnp.maximum(deg, 1.0))
    x0_ref[...] = acc
    dmat_ref[...] = dmat
    y0_ref[...] = _split(acc * dmat)


def _encoder(hm, tables, degp):
    return pl.pallas_call(
        _encoder_body,
        grid=(GRID,),
        in_specs=[
            pl.BlockSpec((BN, NUM_ATOM_FEATS), lambda i: (i, 0)),
            pl.BlockSpec((NUM_ATOM_FEATS, ATOM_VOCAB, D), lambda i: (0, 0, 0)),
            pl.BlockSpec((NC, BN, DH), lambda i: (0, i, 0)),
        ],
        out_specs=[
            pl.BlockSpec((BN, D), lambda i: (i, 0)),
            pl.BlockSpec((NC, BN, DH), lambda i: (0, i, 0)),
            pl.BlockSpec((BN, D), lambda i: (i, 0)),
        ],
        out_shape=[
            jax.ShapeDtypeStruct((N, D), jnp.float32),
            jax.ShapeDtypeStruct((NC, N, DH), jnp.float32),
            jax.ShapeDtypeStruct((N, D), jnp.float32),
        ],
    )(hm, tables, degp)


def _stage_a_body(s1p_ref, dmat_ref, x1_ref, y1_ref):
    dmat = dmat_ref[...]
    s1 = jnp.concatenate([s1p_ref[0], s1p_ref[1]], axis=1)
    x1 = -(s1 * dmat)
    x1_ref[...] = x1
    y1_ref[...] = _split(x1 * dmat)


def _stage_a(s1p, dmat):
    return pl.pallas_call(
        _stage_a_body,
        grid=(GRID,),
        in_specs=[
            pl.BlockSpec((NC, BN, DH), lambda i: (0, i, 0)),
            pl.BlockSpec((BN, D), lambda i: (i, 0)),
        ],
        out_specs=[
            pl.BlockSpec((BN, D), lambda i: (i, 0)),
            pl.BlockSpec((NC, BN, DH), lambda i: (0, i, 0)),
        ],
        out_shape=[
            jax.ShapeDtypeStruct((N, D), jnp.float32),
            jax.ShapeDtypeStruct((NC, N, DH), jnp.float32),
        ],
    )(s1p, dmat)


def _stage_b1_body(s2p_ref, dmat_ref, x0_ref, x1_ref, w_ref, hh_ref, stats_ref):
    i = pl.program_id(0)
    x0 = x0_ref[...]
    s2 = jnp.concatenate([s2p_ref[0], s2p_ref[1]], axis=1)
    x2 = -2.0 * (s2 * dmat_ref[...]) - x0
    hh = (jnp.dot(x0, w_ref[0], preferred_element_type=jnp.float32)
          + jnp.dot(x1_ref[...], w_ref[1], preferred_element_type=jnp.float32)
          + jnp.dot(x2, w_ref[2], preferred_element_type=jnp.float32))
    hh_ref[...] = hh
    ssum = jnp.sum(hh, axis=0, keepdims=True)
    ssq = jnp.sum(hh * hh, axis=0, keepdims=True)
    upd = jnp.concatenate(
        [ssum, ssq, jnp.zeros((6, D), jnp.float32)], axis=0)

    @pl.when(i == 0)
    def _():
        stats_ref[...] = upd

    @pl.when(i > 0)
    def _():
        stats_ref[...] = stats_ref[...] + upd


def _stage_b1(s2p, dmat, x0, x1, wl):
    return pl.pallas_call(
        _stage_b1_body,
        grid=(GRID,),
        in_specs=[
            pl.BlockSpec((NC, BN, DH), lambda i: (0, i, 0)),
            pl.BlockSpec((BN, D), lambda i: (i, 0)),
            pl.BlockSpec((BN, D), lambda i: (i, 0)),
            pl.BlockSpec((BN, D), lambda i: (i, 0)),
            pl.BlockSpec((3, D, D), lambda i: (0, 0, 0)),
        ],
        out_specs=[
            pl.BlockSpec((BN, D), lambda i: (i, 0)),
            pl.BlockSpec((8, D), lambda i: (0, 0)),
        ],
        out_shape=[
            jax.ShapeDtypeStruct((N, D), jnp.float32),
            jax.ShapeDtypeStruct((8, D), jnp.float32),
        ],
    )(s2p, dmat, x0, x1, wl)


def _stage_b2_body(hh_ref, stats_ref, x0_ref, gb_ref, dmat_ref,
                   x_ref, y_ref, xsum_ref):
    i = pl.program_id(0)
    mu = stats_ref[0:1, :] * (1.0 / N)
    var = stats_ref[1:2, :] * (1.0 / N) - mu * mu
    rstd = lax.rsqrt(var + 1e-5)
    hn = (hh_ref[...] - mu) * rstd * gb_ref[0:1, :] + gb_ref[1:2, :]
    x = x0_ref[...] + jnp.maximum(hn, 0.0)
    x_ref[...] = x
    y_ref[...] = _split(x * dmat_ref[...])
    upd = jnp.concatenate(
        [jnp.sum(x, axis=0, keepdims=True), jnp.zeros((7, D), jnp.float32)],
        axis=0)

    @pl.when(i == 0)
    def _():
        xsum_ref[...] = upd

    @pl.when(i > 0)
    def _():
        xsum_ref[...] = xsum_ref[...] + upd


def _stage_b2(hh, stats, x0, gb, dmat):
    return pl.pallas_call(
        _stage_b2_body,
        grid=(GRID,),
        in_specs=[
            pl.BlockSpec((BN, D), lambda i: (i, 0)),
            pl.BlockSpec((8, D), lambda i: (0, 0)),
            pl.BlockSpec((BN, D), lambda i: (i, 0)),
            pl.BlockSpec((8, D), lambda i: (0, 0)),
            pl.BlockSpec((BN, D), lambda i: (i, 0)),
        ],
        out_specs=[
            pl.BlockSpec((BN, D), lambda i: (i, 0)),
            pl.BlockSpec((NC, BN, DH), lambda i: (0, i, 0)),
            pl.BlockSpec((8, D), lambda i: (0, 0)),
        ],
        out_shape=[
            jax.ShapeDtypeStruct((N, D), jnp.float32),
            jax.ShapeDtypeStruct((NC, N, DH), jnp.float32),
            jax.ShapeDtypeStruct((8, D), jnp.float32),
        ],
    )(hh, stats, x0, gb, dmat)


def _readout_body(xsum_ref, w1_ref, b1_ref, w2_ref, b2_ref, w3_ref, b3_ref,
                  y_ref):
    hg = xsum_ref[0:1, :] * (1.0 / N)
    y1 = jnp.maximum(
        jnp.dot(hg, w1_ref[...], preferred_element_type=jnp.float32)
        + b1_ref[...], 0.0)
    y2 = jnp.maximum(
        jnp.dot(y1, w2_ref[...], preferred_element_type=jnp.float32)
        + b2_ref[...], 0.0)
    y_ref[...] = (jnp.dot(y2, w3_ref[...], preferred_element_type=jnp.float32)
                  + b3_ref[...])


def _readout(xsum, w1, b1, w2, b2, w3, b3):
    return pl.pallas_call(
        _readout_body,
        out_shape=jax.ShapeDtypeStruct((1, b3.shape[-1]), jnp.float32),
    )(xsum, w1, b1, w2, b2, w3, b3)


def kernel(h, e, edge_index, atom_tables, bond_tables, W, gamma, beta,
           rW1, rb1, rW2, rb2, rW3, rb3):
    del e, bond_tables
    src = edge_index[0].astype(jnp.int32)
    dst = edge_index[1].astype(jnp.int32)
    sidxp = jnp.pad(src.reshape(NS, EPT),
                    ((0, 0), (0, EPT_A - EPT))).reshape(NS, NCHUNK_A, CH)
    didxp = jnp.pad(dst.reshape(NS, EPT), ((0, 0), (0, EPT_A - EPT)),
                    constant_values=N).reshape(NS, NCHUNK_A, CH)
    zrows = jnp.zeros((ZROWS, DH), jnp.float32)

    degp = _degsum(jnp.ones((CH, DH), jnp.float32), didxp, zrows)
    x0, y, dmat = _encoder(h.astype(jnp.int32), atom_tables, degp)

    xsum = None
    for l in range(NUM_LAYERS):
        s1p = _segsum(y, sidxp, didxp, zrows)
        x1, y1 = _stage_a(s1p, dmat)
        s2p = _segsum(y1, sidxp, didxp, zrows)
        hh, stats = _stage_b1(s2p, dmat, x0, x1, W[l].reshape(3, D, D))
        gb = jnp.concatenate(
            [gamma[l].reshape(1, D), beta[l].reshape(1, D),
             jnp.zeros((6, D), jnp.float32)], axis=0)
        x0, y, xsum = _stage_b2(hh, stats, x0, gb, dmat)

    return _readout(xsum, rW1, rb1.reshape(1, -1), rW2, rb2.reshape(1, -1),
                    rW3, rb3.reshape(1, -1))
